# baseline (speedup 1.0000x reference)
import functools

import jax
import jax.numpy as jnp
from jax import lax
from jax.experimental import pallas as pl
from jax.experimental.pallas import tpu as pltpu
from jax.experimental.pallas import tpu_sc as plsc

_NW = 32
_CH = 32
_SIZES = (2048, 1024, 512, 256, 128, 64, 32, 16, 8)


def _sc_fill_body(x, mt, kk, out, kv, fill_v, buf, gsem, fsem, flsem):
    B, S, D = x.shape
    nch = (B * S) // (_NW * _CH)
    cpb = S // _CH

    cid = lax.axis_index("c")
    sid = lax.axis_index("s")
    wid = sid * 2 + cid

    pltpu.make_async_copy(mt, fill_v, flsem).start()

    pltpu.sync_copy(kk, kv.at[pl.ds(0, B)])
    vals = kv[...]

    def chunk_info(ci):
        g = wid + ci * _NW
        b = g // cpb
        s0 = (g % cpb) * _CH
        k_b = vals[0]
        for bb in range(1, B):
            k_b = jnp.where(b == bb, vals[bb], k_b)
        kept = jnp.clip(k_b - s0, 0, _CH)
        return b, s0, kept

    pltpu.make_async_copy(mt, fill_v, flsem).wait()

    def visit(ci, carry):
        b, s0, kept = chunk_info(ci)

        @pl.when(kept == 0)
        def _fill():
            pltpu.make_async_copy(fill_v, out.at[b, pl.ds(s0, _CH)], fsem).start()

        @pl.when((kept > 0) & (kept < _CH))
        def _partial():
            pltpu.async_copy(x.at[b, pl.ds(s0, _CH)], buf, gsem).wait()

            def fire(r, c):
                @pl.when(r < kept)
                def _row_keep():
                    pltpu.make_async_copy(
                        buf.at[pl.ds(r, 1)],
                        out.at[b, pl.ds(s0 + r, 1)], fsem).start()

                @pl.when(r >= kept)
                def _row_drop():
                    pltpu.make_async_copy(
                        fill_v.at[pl.ds(r, 1)],
                        out.at[b, pl.ds(s0 + r, 1)], fsem).start()
                return c

            def drain(r, c):
                pltpu.make_async_copy(
                    fill_v.at[pl.ds(r, 1)],
                    out.at[b, pl.ds(s0 + r, 1)], fsem).wait()
                return c

            lax.fori_loop(0, _CH, fire, 0)
            lax.fori_loop(0, _CH, drain, 0)
        return carry

    lax.fori_loop(0, nch, visit, 0)

    def drain_fill(ci, carry):
        b, s0, kept = chunk_info(ci)

        @pl.when(kept == 0)
        def _drain():
            pltpu.make_async_copy(fill_v, out.at[b, pl.ds(s0, _CH)], fsem).wait()
        return carry

    lax.fori_loop(0, nch, drain_fill, 0)


def _tc_copy_body(o1_ref, x_ref, keep_ref, o_ref, sem):
    B, S, D = x_ref.shape
    del o1_ref

    def passes(fire):
        for b in range(B):
            off = jnp.int32(0)
            rem = (keep_ref[b] // 8) * 8
            for sz in _SIZES:
                if sz > S:
                    continue
                cond = rem >= sz
                off_c = pl.multiple_of(off, 8)

                @pl.when(cond)
                def _():
                    c = pltpu.make_async_copy(
                        x_ref.at[b, pl.ds(off_c, sz)],
                        o_ref.at[b, pl.ds(off_c, sz)], sem)
                    if fire:
                        c.start()
                    else:
                        c.wait()

                off = jnp.where(cond, off + sz, off)
                rem = jnp.where(cond, rem - sz, rem)

    passes(True)
    passes(False)


def kernel(x, mask_token, keep_k):
    B, S, D = x.shape
    mask_block = jnp.tile(mask_token[None, :], (_CH, 1))

    sc_fill = functools.partial(
        pl.kernel,
        out_type=jax.ShapeDtypeStruct((B, S, D), x.dtype),
        mesh=plsc.VectorSubcoreMesh(core_axis_name="c", subcore_axis_name="s"),
        scratch_types=[
            pltpu.VMEM((16,), jnp.int32),
            pltpu.VMEM((_CH, D), x.dtype),
            pltpu.VMEM((_CH, D), x.dtype),
            pltpu.SemaphoreType.DMA,
            pltpu.SemaphoreType.DMA,
            pltpu.SemaphoreType.DMA,
        ],
    )(_sc_fill_body)
    out1 = sc_fill(x, mask_block, keep_k)

    tc_copy = pl.pallas_call(
        _tc_copy_body,
        grid=(),
        in_specs=[
            pl.BlockSpec(memory_space=pltpu.MemorySpace.HBM),
            pl.BlockSpec(memory_space=pltpu.MemorySpace.HBM),
            pl.BlockSpec(memory_space=pltpu.MemorySpace.SMEM),
        ],
        out_specs=pl.BlockSpec(memory_space=pltpu.MemorySpace.HBM),
        out_shape=jax.ShapeDtypeStruct((B, S, D), x.dtype),
        scratch_shapes=[pltpu.SemaphoreType.DMA],
        input_output_aliases={0: 0},
    )
    return tc_copy(out1, x, keep_k)


# speedup vs baseline: 14.9395x; 14.9395x over previous
import functools

import jax
import jax.numpy as jnp
from jax import lax
from jax.experimental import pallas as pl
from jax.experimental.pallas import tpu as pltpu
from jax.experimental.pallas import tpu_sc as plsc

_NW = 32
_CH = 32
_SIZES = (2048, 1024, 512, 256, 128, 64, 32, 16, 8)


def _sc_fill_body(x, mt, kk, out, kv, fill_v, buf, gsem, fsem, flsem):
    B, S, D = x.shape
    nch = (B * S) // (_NW * _CH)
    cpb = S // _CH

    cid = lax.axis_index("c")
    sid = lax.axis_index("s")
    wid = sid * 2 + cid

    pltpu.make_async_copy(mt, fill_v, flsem).start()

    pltpu.sync_copy(kk, kv.at[pl.ds(0, B)])
    vals = kv[...]

    def chunk_info(ci):
        g = wid + ci * _NW
        b = g // cpb
        s0 = (g % cpb) * _CH
        k_b = vals[0]
        for bb in range(1, B):
            k_b = jnp.where(b == bb, vals[bb], k_b)
        kept = jnp.clip(k_b - s0, 0, _CH)
        return b, s0, kept

    pltpu.make_async_copy(mt, fill_v, flsem).wait()

    def visit(ci, carry):
        b, s0, kept = chunk_info(ci)

        @pl.when(kept == 0)
        def _fill():
            pltpu.make_async_copy(fill_v, out.at[b, pl.ds(s0, _CH)], fsem).start()

        @pl.when((kept > 0) & (kept < _CH))
        def _partial():
            pltpu.async_copy(x.at[b, pl.ds(s0, _CH)], buf, gsem).wait()

            def fire(r, c):
                @pl.when(r < kept)
                def _row_keep():
                    pltpu.make_async_copy(
                        buf.at[pl.ds(r, 1)],
                        out.at[b, pl.ds(s0 + r, 1)], fsem).start()

                @pl.when(r >= kept)
                def _row_drop():
                    pltpu.make_async_copy(
                        fill_v.at[pl.ds(r, 1)],
                        out.at[b, pl.ds(s0 + r, 1)], fsem).start()
                return c

            def drain(r, c):
                pltpu.make_async_copy(
                    fill_v.at[pl.ds(r, 1)],
                    out.at[b, pl.ds(s0 + r, 1)], fsem).wait()
                return c

            lax.fori_loop(0, _CH, fire, 0)
            lax.fori_loop(0, _CH, drain, 0)
        return carry

    lax.fori_loop(0, nch, visit, 0)

    def drain_fill(ci, carry):
        b, s0, kept = chunk_info(ci)

        @pl.when(kept == 0)
        def _drain():
            pltpu.make_async_copy(fill_v, out.at[b, pl.ds(s0, _CH)], fsem).wait()
        return carry

    lax.fori_loop(0, nch, drain_fill, 0)


def _tc_copy_body(o1_ref, x_ref, keep_ref, o_ref, *rest):
    B, S, D = x_ref.shape
    del o1_ref
    nsz = len(_SIZES)
    bufs = rest[:2 * nsz]
    gsem, ssem = rest[2 * nsz:]

    def ak(b):
        return (keep_ref[b] // 8) * 8

    def cond(si, b):
        return (ak(b) & _SIZES[si]) != 0

    def off(si, b):
        sz = _SIZES[si]
        return pl.multiple_of(ak(b) & ~(2 * sz - 1), 8)

    def gather(si, b):
        sz = _SIZES[si]
        return pltpu.make_async_copy(
            x_ref.at[b, pl.ds(off(si, b), sz)],
            bufs[2 * si + b % 2], gsem.at[si, b % 2])

    def scatter(si, b):
        sz = _SIZES[si]
        return pltpu.make_async_copy(
            bufs[2 * si + b % 2],
            o_ref.at[b, pl.ds(off(si, b), sz)], ssem.at[si, b % 2])

    def gstart(si, b):
        @pl.when(cond(si, b))
        def _():
            gather(si, b).start()

    for si in range(nsz):
        gstart(si, 0)
    for b in range(B):
        for si in range(nsz):
            if b + 1 < B:
                if b >= 1:
                    @pl.when(cond(si, b - 1))
                    def _release():
                        scatter(si, b - 1).wait()
                gstart(si, b + 1)

            @pl.when(cond(si, b))
            def _move():
                gather(si, b).wait()
                scatter(si, b).start()
    for si in range(nsz):
        for b in (B - 2, B - 1):
            @pl.when(cond(si, b))
            def _drain():
                scatter(si, b).wait()


def kernel(x, mask_token, keep_k):
    B, S, D = x.shape
    mask_block = jnp.tile(mask_token[None, :], (_CH, 1))

    sc_fill = functools.partial(
        pl.kernel,
        out_type=jax.ShapeDtypeStruct((B, S, D), x.dtype),
        mesh=plsc.VectorSubcoreMesh(core_axis_name="c", subcore_axis_name="s"),
        scratch_types=[
            pltpu.VMEM((16,), jnp.int32),
            pltpu.VMEM((_CH, D), x.dtype),
            pltpu.VMEM((_CH, D), x.dtype),
            pltpu.SemaphoreType.DMA,
            pltpu.SemaphoreType.DMA,
            pltpu.SemaphoreType.DMA,
        ],
    )(_sc_fill_body)
    out1 = sc_fill(x, mask_block, keep_k)

    nsz = len(_SIZES)
    buf_shapes = []
    for sz in _SIZES:
        buf_shapes.append(pltpu.VMEM((sz, D), x.dtype))
        buf_shapes.append(pltpu.VMEM((sz, D), x.dtype))
    tc_copy = pl.pallas_call(
        _tc_copy_body,
        grid=(),
        in_specs=[
            pl.BlockSpec(memory_space=pltpu.MemorySpace.HBM),
            pl.BlockSpec(memory_space=pltpu.MemorySpace.HBM),
            pl.BlockSpec(memory_space=pltpu.MemorySpace.SMEM),
        ],
        out_specs=pl.BlockSpec(memory_space=pltpu.MemorySpace.HBM),
        out_shape=jax.ShapeDtypeStruct((B, S, D), x.dtype),
        scratch_shapes=buf_shapes + [
            pltpu.SemaphoreType.DMA((nsz, 2)),
            pltpu.SemaphoreType.DMA((nsz, 2)),
        ],
        input_output_aliases={0: 0},
    )
    return tc_copy(out1, x, keep_k)
